# baseline (speedup 1.0000x reference)
import jax
import jax.numpy as jnp
from jax.experimental import pallas as pl
from jax.experimental.pallas import tpu as pltpu

BEAT_LEN = 128
TOKEN_DIM = 64
N_CHUNKS = 2


def _proj_kernel(x_hbm, wt_ref, b_ref, o_hbm, bi_ref, x_vmem, y_vmem,
                 in_sems, out_sems):
    B, L, T = x_hbm.shape
    N = T // BEAT_LEN
    C = B // N_CHUNKS

    pltpu.make_async_copy(
        x_hbm.at[0:C], x_vmem.at[0:C], in_sems.at[0]
    ).start()

    wt = wt_ref[...]
    bias = b_ref[...]
    bi_ref[...] = jnp.full(bi_ref.shape, float(BEAT_LEN), dtype=jnp.float32)

    for c in range(N_CHUNKS):
        lo, hi = c * C, (c + 1) * C
        pltpu.make_async_copy(
            x_hbm.at[lo:hi], x_vmem.at[lo:hi], in_sems.at[c]
        ).wait()
        if c + 1 < N_CHUNKS:
            nlo, nhi = (c + 1) * C, (c + 2) * C
            pltpu.make_async_copy(
                x_hbm.at[nlo:nhi], x_vmem.at[nlo:nhi], in_sems.at[c + 1]
            ).start()
        x = x_vmem[lo:hi].reshape(C * L * N, BEAT_LEN)
        y = jnp.dot(x, wt, preferred_element_type=jnp.float32) + bias
        y_vmem[lo:hi] = y.reshape(C, L, N, TOKEN_DIM)
        pltpu.make_async_copy(
            y_vmem.at[lo:hi], o_hbm.at[lo:hi], out_sems.at[c]
        ).start()

    for c in range(N_CHUNKS):
        lo, hi = c * C, (c + 1) * C
        pltpu.make_async_copy(
            y_vmem.at[lo:hi], o_hbm.at[lo:hi], out_sems.at[c]
        ).wait()


@jax.jit
def _run(ecg, W, b):
    B, L, T = ecg.shape
    N = T // BEAT_LEN
    wt = W.T
    b2 = b.reshape(1, TOKEN_DIM)

    X, bi = pl.pallas_call(
        _proj_kernel,
        in_specs=[
            pl.BlockSpec(memory_space=pl.ANY),
            pl.BlockSpec((BEAT_LEN, TOKEN_DIM), lambda: (0, 0)),
            pl.BlockSpec((1, TOKEN_DIM), lambda: (0, 0)),
        ],
        out_specs=[
            pl.BlockSpec(memory_space=pl.ANY),
            pl.BlockSpec((B, N), lambda: (0, 0)),
        ],
        out_shape=[
            jax.ShapeDtypeStruct((B, L, N, TOKEN_DIM), jnp.float32),
            jax.ShapeDtypeStruct((B, N), jnp.float32),
        ],
        scratch_shapes=[
            pltpu.VMEM((B, L, T), jnp.float32),
            pltpu.VMEM((B, L, N, TOKEN_DIM), jnp.float32),
            pltpu.SemaphoreType.DMA((N_CHUNKS,)),
            pltpu.SemaphoreType.DMA((N_CHUNKS,)),
        ],
    )(ecg, wt, b2)

    return (X, bi)


def kernel(ecg, W, b):
    return _run(ecg, W, b)


# speedup vs baseline: 1.1948x; 1.1948x over previous
import jax
import jax.numpy as jnp
from jax.experimental import pallas as pl
from jax.experimental.pallas import tpu as pltpu

BEAT_LEN = 128
TOKEN_DIM = 64


def _proj_kernel(x_ref, wt_ref, b_ref, o_ref, bi_ref):
    o_ref[...] = (
        jnp.dot(x_ref[...], wt_ref[...], preferred_element_type=jnp.float32)
        + b_ref[...]
    )
    bi_ref[...] = jnp.full(bi_ref.shape, float(BEAT_LEN), dtype=jnp.float32)


@jax.jit
def _run(ecg, W, b):
    B, L, T = ecg.shape
    N = T // BEAT_LEN
    M = B * L * N
    x = ecg.reshape(M, BEAT_LEN)
    wt = W.T
    b2 = b.reshape(1, TOKEN_DIM)

    out, bi = pl.pallas_call(
        _proj_kernel,
        in_specs=[
            pl.BlockSpec((M, BEAT_LEN), lambda: (0, 0)),
            pl.BlockSpec((BEAT_LEN, TOKEN_DIM), lambda: (0, 0)),
            pl.BlockSpec((1, TOKEN_DIM), lambda: (0, 0)),
        ],
        out_specs=[
            pl.BlockSpec((M, TOKEN_DIM), lambda: (0, 0)),
            pl.BlockSpec((B, N), lambda: (0, 0)),
        ],
        out_shape=[
            jax.ShapeDtypeStruct((M, TOKEN_DIM), jnp.float32),
            jax.ShapeDtypeStruct((B, N), jnp.float32),
        ],
        compiler_params=pltpu.CompilerParams(
            allow_input_fusion=[True, False, False],
        ),
    )(x, wt, b2)

    X = out.reshape(B, L, N, TOKEN_DIM)
    return (X, bi)


def kernel(ecg, W, b):
    return _run(ecg, W, b)
